# initial kernel scaffold (unmeasured)
import jax
import jax.numpy as jnp
from jax import lax
from jax.experimental import pallas as pl
from jax.experimental.pallas import tpu as pltpu

N_DEV = 4
SCALE = 0.08838834764831843


def kernel(x, Wq, Wo, K_ext, V_ext):
    B, Sq, D = x.shape
    Dq = Wq.shape[1]
    _, Skv, Hloc, Dh = K_ext.shape

    def body(x_ref, wq_ref, wo_ref, k_ref, v_ref, out_ref,
             comm_ref, send_sems, recv_sems):
        my = lax.axis_index("i")
        left = lax.rem(my + N_DEV - 1, N_DEV)
        right = lax.rem(my + 1, N_DEV)

        barrier_sem = pltpu.get_barrier_semaphore()
        for nbr in (left, right):
            pl.semaphore_signal(barrier_sem, inc=1, device_id=(nbr,),
                                device_id_type=pl.DeviceIdType.MESH)
        pl.semaphore_wait(barrier_sem, 2)

        wq = wq_ref[...].astype(jnp.bfloat16)
        wo = wo_ref[...].astype(jnp.bfloat16)
        for b in range(B):
            xb = x_ref[b].astype(jnp.bfloat16)
            qb = jnp.dot(xb, wq, preferred_element_type=jnp.float32)
            qb = qb.astype(jnp.bfloat16)
            kb = k_ref[b][...].reshape(Skv, Hloc * Dh).astype(jnp.bfloat16)
            vb = v_ref[b][...].reshape(Skv, Hloc * Dh).astype(jnp.bfloat16)
            heads = []
            for h in range(Hloc):
                qh = qb[:, h * Dh:(h + 1) * Dh]
                kh = kb[:, h * Dh:(h + 1) * Dh]
                vh = vb[:, h * Dh:(h + 1) * Dh]
                s = lax.dot_general(qh, kh, (((1,), (1,)), ((), ())),
                                    preferred_element_type=jnp.float32)
                s = s * SCALE
                m = jnp.max(s, axis=1, keepdims=True)
                p = jnp.exp(s - m)
                l = jnp.sum(p, axis=1, keepdims=True)
                o = jnp.dot(p.astype(jnp.bfloat16), vh,
                            preferred_element_type=jnp.float32)
                heads.append((o / l).astype(jnp.bfloat16))
            ab = jnp.concatenate(heads, axis=1)
            pb = jnp.dot(ab, wo, preferred_element_type=jnp.float32)
            out_ref[b] = pb
            comm_ref[0, b] = pb.astype(jnp.bfloat16)

        for hop in range(N_DEV - 1):
            rdma = pltpu.make_async_remote_copy(
                src_ref=comm_ref.at[hop],
                dst_ref=comm_ref.at[hop + 1],
                send_sem=send_sems.at[hop],
                recv_sem=recv_sems.at[hop],
                device_id=(right,),
                device_id_type=pl.DeviceIdType.MESH,
            )
            rdma.start()
            rdma.wait()
            out_ref[...] = out_ref[...] + comm_ref[hop + 1].astype(jnp.float32)

    return pl.pallas_call(
        body,
        out_shape=jax.ShapeDtypeStruct((B, Sq, D), jnp.float32),
        in_specs=[pl.BlockSpec(memory_space=pltpu.VMEM)] * 5,
        out_specs=pl.BlockSpec(memory_space=pltpu.VMEM),
        scratch_shapes=[
            pltpu.VMEM((N_DEV, B, Sq, D), jnp.bfloat16),
            pltpu.SemaphoreType.DMA((N_DEV - 1,)),
            pltpu.SemaphoreType.DMA((N_DEV - 1,)),
        ],
        compiler_params=pltpu.CompilerParams(collective_id=0),
    )(x, Wq, Wo, K_ext, V_ext)


# baseline (device time: 117345 ns/iter reference)
import jax
import jax.numpy as jnp
from jax import lax
from jax.experimental import pallas as pl
from jax.experimental.pallas import tpu as pltpu

N_DEV = 4
SCALE = 0.08838834764831843


def kernel(x, Wq, Wo, K_ext, V_ext):
    B, Sq, D = x.shape
    Dq = Wq.shape[1]
    _, Skv, Hloc, Dh = K_ext.shape

    def body(x_ref, wq_ref, wo_ref, k_hbm, v_hbm, out_ref,
             comm_ref, k_stage, v_stage, kv_sems, send_sems, recv_sems):
        my = lax.axis_index("i")
        left = lax.rem(my + N_DEV - 1, N_DEV)
        right = lax.rem(my + 1, N_DEV)

        barrier_sem = pltpu.get_barrier_semaphore()
        for nbr in (left, right):
            pl.semaphore_signal(barrier_sem, inc=1, device_id=(nbr,),
                                device_id_type=pl.DeviceIdType.MESH)
        pl.semaphore_wait(barrier_sem, 2)

        def kv_copies(b, slot):
            ck = pltpu.make_async_copy(
                k_hbm.at[b], k_stage.at[slot], kv_sems.at[slot, 0])
            cv = pltpu.make_async_copy(
                v_hbm.at[b], v_stage.at[slot], kv_sems.at[slot, 1])
            return ck, cv

        wq = wq_ref[...].astype(jnp.bfloat16)
        wo = wo_ref[...].astype(jnp.bfloat16)
        ck, cv = kv_copies(0, 0)
        ck.start()
        cv.start()
        for b in range(B):
            slot = b % 2
            if b + 1 < B:
                nk, nv = kv_copies(b + 1, 1 - slot)
                nk.start()
                nv.start()
            ck, cv = kv_copies(b, slot)
            ck.wait()
            cv.wait()
            xb = x_ref[b].astype(jnp.bfloat16)
            qb = jnp.dot(xb, wq, preferred_element_type=jnp.float32)
            qb = qb.astype(jnp.bfloat16)
            kb = k_stage[slot].reshape(Skv, Hloc * Dh).astype(jnp.bfloat16)
            vb = v_stage[slot].reshape(Skv, Hloc * Dh).astype(jnp.bfloat16)
            heads = []
            for h in range(Hloc):
                qh = qb[:, h * Dh:(h + 1) * Dh]
                kh = kb[:, h * Dh:(h + 1) * Dh]
                vh = vb[:, h * Dh:(h + 1) * Dh]
                s = lax.dot_general(qh, kh, (((1,), (1,)), ((), ())),
                                    preferred_element_type=jnp.float32)
                s = s * SCALE
                m = jnp.max(s, axis=1, keepdims=True)
                p = jnp.exp(s - m)
                l = jnp.sum(p, axis=1, keepdims=True)
                o = jnp.dot(p.astype(jnp.bfloat16), vh,
                            preferred_element_type=jnp.float32)
                heads.append((o / l).astype(jnp.bfloat16))
            ab = jnp.concatenate(heads, axis=1)
            pb = jnp.dot(ab, wo, preferred_element_type=jnp.float32)
            out_ref[b] = pb
            comm_ref[0, b] = pb.astype(jnp.bfloat16)

        for hop in range(N_DEV - 1):
            rdma = pltpu.make_async_remote_copy(
                src_ref=comm_ref.at[hop],
                dst_ref=comm_ref.at[hop + 1],
                send_sem=send_sems.at[hop],
                recv_sem=recv_sems.at[hop],
                device_id=(right,),
                device_id_type=pl.DeviceIdType.MESH,
            )
            rdma.start()
            rdma.wait()
            out_ref[...] = out_ref[...] + comm_ref[hop + 1].astype(jnp.float32)

    return pl.pallas_call(
        body,
        out_shape=jax.ShapeDtypeStruct((B, Sq, D), jnp.float32),
        in_specs=[
            pl.BlockSpec(memory_space=pltpu.VMEM),
            pl.BlockSpec(memory_space=pltpu.VMEM),
            pl.BlockSpec(memory_space=pltpu.VMEM),
            pl.BlockSpec(memory_space=pltpu.HBM),
            pl.BlockSpec(memory_space=pltpu.HBM),
        ],
        out_specs=pl.BlockSpec(memory_space=pltpu.VMEM),
        scratch_shapes=[
            pltpu.VMEM((N_DEV, B, Sq, D), jnp.bfloat16),
            pltpu.VMEM((2, Skv, Hloc, Dh), jnp.float32),
            pltpu.VMEM((2, Skv, Hloc, Dh), jnp.float32),
            pltpu.SemaphoreType.DMA((2, 2)),
            pltpu.SemaphoreType.DMA((N_DEV - 1,)),
            pltpu.SemaphoreType.DMA((N_DEV - 1,)),
        ],
        compiler_params=pltpu.CompilerParams(
            collective_id=0,
            vmem_limit_bytes=100 * 1024 * 1024,
        ),
    )(x, Wq, Wo, K_ext, V_ext)


# device time: 68322 ns/iter; 1.7175x vs baseline; 1.7175x over previous
import jax
import jax.numpy as jnp
from jax import lax
from jax.experimental import pallas as pl
from jax.experimental.pallas import tpu as pltpu

N_DEV = 4
SCALE = 0.08838834764831843


def kernel(x, Wq, Wo, K_ext, V_ext):
    B, Sq, D = x.shape
    _, Skv, Hloc, Dh = K_ext.shape

    def body(x_ref, wq_ref, wo_ref, k_hbm, v_hbm, out_ref,
             send_buf, recv_buf, k_stage, v_stage,
             kv_sems, send_sems, recv_sems):
        my = lax.axis_index("i")
        left = lax.rem(my + N_DEV - 1, N_DEV)
        right = lax.rem(my + 1, N_DEV)

        barrier_sem = pltpu.get_barrier_semaphore()
        for nbr in (left, right):
            pl.semaphore_signal(barrier_sem, inc=1, device_id=(nbr,),
                                device_id_type=pl.DeviceIdType.MESH)
        pl.semaphore_wait(barrier_sem, 2)

        def kv_copies(b, slot):
            ck = pltpu.make_async_copy(
                k_hbm.at[b], k_stage.at[slot], kv_sems.at[slot, 0])
            cv = pltpu.make_async_copy(
                v_hbm.at[b], v_stage.at[slot], kv_sems.at[slot, 1])
            return ck, cv

        def ring_rdma(r, src_ref):
            return pltpu.make_async_remote_copy(
                src_ref=src_ref,
                dst_ref=recv_buf.at[r],
                send_sem=send_sems.at[r],
                recv_sem=recv_sems.at[r],
                device_id=(right,),
                device_id_type=pl.DeviceIdType.MESH,
            )

        wq = wq_ref[...].astype(jnp.bfloat16)
        wo = wo_ref[...].astype(jnp.bfloat16)

        def batch_at(t):
            return lax.rem(my - t + 2 * N_DEV, N_DEV)

        b0 = batch_at(0)
        ck, cv = kv_copies(b0, 0)
        ck.start()
        cv.start()

        def compute_partial(b, slot):
            xb = x_ref[b].astype(jnp.bfloat16)
            qb = jnp.dot(xb, wq, preferred_element_type=jnp.float32)
            qb = qb.astype(jnp.bfloat16)
            kb = k_stage[slot].reshape(Skv, Hloc * Dh).astype(jnp.bfloat16)
            vb = v_stage[slot].reshape(Skv, Hloc * Dh).astype(jnp.bfloat16)
            heads = []
            for h in range(Hloc):
                qh = qb[:, h * Dh:(h + 1) * Dh]
                kh = kb[:, h * Dh:(h + 1) * Dh]
                vh = vb[:, h * Dh:(h + 1) * Dh]
                s = lax.dot_general(qh, kh, (((1,), (1,)), ((), ())),
                                    preferred_element_type=jnp.float32)
                s = s * SCALE
                m = jnp.max(s, axis=1, keepdims=True)
                p = jnp.exp(s - m)
                l = jnp.sum(p, axis=1, keepdims=True)
                o = jnp.dot(p.astype(jnp.bfloat16), vh,
                            preferred_element_type=jnp.float32)
                heads.append((o / l).astype(jnp.bfloat16))
            ab = jnp.concatenate(heads, axis=1)
            return jnp.dot(ab, wo, preferred_element_type=jnp.float32)

        if B > 1:
            nb, nv = kv_copies(batch_at(1), 1)
            nb.start()
            nv.start()
        ck, cv = kv_copies(b0, 0)
        ck.wait()
        cv.wait()
        p0 = compute_partial(b0, 0)
        out_ref[b0] = p0
        send_buf[0] = p0.astype(jnp.bfloat16)
        rs = [None] * (N_DEV - 1)
        rs[0] = ring_rdma(0, send_buf.at[0])
        rs[0].start()

        for t in range(1, N_DEV):
            slot = t % 2
            if t + 1 < N_DEV:
                nk, nv = kv_copies(batch_at(t + 1), 1 - slot)
                nk.start()
                nv.start()
            ck, cv = kv_copies(batch_at(t), slot)
            ck.wait()
            cv.wait()
            bt = batch_at(t)
            pt = compute_partial(bt, slot)
            out_ref[bt] = pt
            rs[t - 1].wait_recv()
            acc = recv_buf[t - 1].astype(jnp.float32) + pt
            if t < N_DEV - 1:
                send_buf[t] = acc.astype(jnp.bfloat16)
                rs[t] = ring_rdma(t, send_buf.at[t])
                rs[t].start()
            else:
                out_ref[bt] = acc
                send_buf[N_DEV - 1] = acc.astype(jnp.bfloat16)

        ag = [None] * (N_DEV - 1)
        ag[0] = ring_rdma(N_DEV - 1, send_buf.at[N_DEV - 1])
        ag[0].start()
        for s in range(N_DEV - 1):
            ag[s].wait_recv()
            r = N_DEV - 1 + s
            c = lax.rem(my - s + 2 * N_DEV, N_DEV)
            if s + 1 < N_DEV - 1:
                ag[s + 1] = ring_rdma(r + 1, recv_buf.at[r])
                ag[s + 1].start()
            out_ref[c] = recv_buf[r].astype(jnp.float32)

        for r, d in enumerate(rs + ag):
            d.wait_send()

    n_rdma = 2 * (N_DEV - 1)
    return pl.pallas_call(
        body,
        out_shape=jax.ShapeDtypeStruct((B, Sq, D), jnp.float32),
        in_specs=[
            pl.BlockSpec(memory_space=pltpu.VMEM),
            pl.BlockSpec(memory_space=pltpu.VMEM),
            pl.BlockSpec(memory_space=pltpu.VMEM),
            pl.BlockSpec(memory_space=pltpu.HBM),
            pl.BlockSpec(memory_space=pltpu.HBM),
        ],
        out_specs=pl.BlockSpec(memory_space=pltpu.VMEM),
        scratch_shapes=[
            pltpu.VMEM((N_DEV, Sq, D), jnp.bfloat16),
            pltpu.VMEM((n_rdma, Sq, D), jnp.bfloat16),
            pltpu.VMEM((2, Skv, Hloc, Dh), jnp.float32),
            pltpu.VMEM((2, Skv, Hloc, Dh), jnp.float32),
            pltpu.SemaphoreType.DMA((2, 2)),
            pltpu.SemaphoreType.DMA((n_rdma,)),
            pltpu.SemaphoreType.DMA((n_rdma,)),
        ],
        compiler_params=pltpu.CompilerParams(
            collective_id=0,
            vmem_limit_bytes=100 * 1024 * 1024,
        ),
    )(x, Wq, Wo, K_ext, V_ext)


# device time: 61408 ns/iter; 1.9109x vs baseline; 1.1126x over previous
import jax
import jax.numpy as jnp
from jax import lax
from jax.experimental import pallas as pl
from jax.experimental.pallas import tpu as pltpu

N_DEV = 4
SCALE = 0.08838834764831843


def kernel(x, Wq, Wo, K_ext, V_ext):
    B, Sq, D = x.shape
    _, Skv, Hloc, Dh = K_ext.shape

    def body(x_ref, wq_ref, wo_ref, k_hbm, v_hbm, out_ref,
             send_buf, recv_buf, k_stage, v_stage,
             kv_sems, send_sems, recv_sems):
        my = lax.axis_index("i")
        left = lax.rem(my + N_DEV - 1, N_DEV)
        right = lax.rem(my + 1, N_DEV)

        barrier_sem = pltpu.get_barrier_semaphore()
        for nbr in (left, right):
            pl.semaphore_signal(barrier_sem, inc=1, device_id=(nbr,),
                                device_id_type=pl.DeviceIdType.MESH)
        pl.semaphore_wait(barrier_sem, 2)

        def kv_copies(b, slot):
            ck = pltpu.make_async_copy(
                k_hbm.at[b], k_stage.at[slot], kv_sems.at[slot, 0])
            cv = pltpu.make_async_copy(
                v_hbm.at[b], v_stage.at[slot], kv_sems.at[slot, 1])
            return ck, cv

        def ring_rdma(r, src_ref, to=None):
            return pltpu.make_async_remote_copy(
                src_ref=src_ref,
                dst_ref=recv_buf.at[r],
                send_sem=send_sems.at[r],
                recv_sem=recv_sems.at[r],
                device_id=(right if to is None else to,),
                device_id_type=pl.DeviceIdType.MESH,
            )

        wq = wq_ref[...].astype(jnp.bfloat16)
        wo = wo_ref[...].astype(jnp.bfloat16)

        def batch_at(t):
            return lax.rem(my - t + 2 * N_DEV, N_DEV)

        b0 = batch_at(0)
        ck, cv = kv_copies(b0, 0)
        ck.start()
        cv.start()

        def compute_partial(b, slot):
            xb = x_ref[b].astype(jnp.bfloat16)
            qb = jnp.dot(xb, wq, preferred_element_type=jnp.float32)
            qb = qb.astype(jnp.bfloat16)
            kb = k_stage[slot].reshape(Skv, Hloc * Dh).astype(jnp.bfloat16)
            vb = v_stage[slot].reshape(Skv, Hloc * Dh).astype(jnp.bfloat16)
            heads = []
            for h in range(Hloc):
                qh = qb[:, h * Dh:(h + 1) * Dh]
                kh = kb[:, h * Dh:(h + 1) * Dh]
                vh = vb[:, h * Dh:(h + 1) * Dh]
                s = lax.dot_general(qh, kh, (((1,), (1,)), ((), ())),
                                    preferred_element_type=jnp.float32)
                s = s * SCALE
                m = jnp.max(s, axis=1, keepdims=True)
                p = jnp.exp(s - m)
                l = jnp.sum(p, axis=1, keepdims=True)
                o = jnp.dot(p.astype(jnp.bfloat16), vh,
                            preferred_element_type=jnp.float32)
                heads.append((o / l).astype(jnp.bfloat16))
            ab = jnp.concatenate(heads, axis=1)
            return jnp.dot(ab, wo, preferred_element_type=jnp.float32)

        if B > 1:
            nb, nv = kv_copies(batch_at(1), 1)
            nb.start()
            nv.start()
        ck, cv = kv_copies(b0, 0)
        ck.wait()
        cv.wait()
        p0 = compute_partial(b0, 0)
        out_ref[b0] = p0
        send_buf[0] = p0.astype(jnp.bfloat16)
        rs = [None] * (N_DEV - 1)
        rs[0] = ring_rdma(0, send_buf.at[0])
        rs[0].start()

        for t in range(1, N_DEV):
            slot = t % 2
            if t + 1 < N_DEV:
                nk, nv = kv_copies(batch_at(t + 1), 1 - slot)
                nk.start()
                nv.start()
            ck, cv = kv_copies(batch_at(t), slot)
            ck.wait()
            cv.wait()
            bt = batch_at(t)
            pt = compute_partial(bt, slot)
            out_ref[bt] = pt
            rs[t - 1].wait_recv()
            acc = recv_buf[t - 1].astype(jnp.float32) + pt
            if t < N_DEV - 1:
                send_buf[t] = acc.astype(jnp.bfloat16)
                rs[t] = ring_rdma(t, send_buf.at[t])
                rs[t].start()
            else:
                out_ref[bt] = acc
                send_buf[N_DEV - 1] = acc.astype(jnp.bfloat16)

        ag0r = ring_rdma(3, send_buf.at[N_DEV - 1])
        ag0l = ring_rdma(4, send_buf.at[N_DEV - 1], to=left)
        ag0r.start()
        ag0l.start()
        ag0r.wait_recv()
        ag1 = ring_rdma(5, recv_buf.at[3])
        ag1.start()
        out_ref[my] = recv_buf[3].astype(jnp.float32)
        ag0l.wait_recv()
        out_ref[lax.rem(my + 2, N_DEV)] = recv_buf[4].astype(jnp.float32)
        ag1.wait_recv()
        out_ref[lax.rem(my + N_DEV - 1, N_DEV)] = recv_buf[5].astype(jnp.float32)
        ag = [ag0r, ag0l, ag1]

        for r, d in enumerate(rs + ag):
            d.wait_send()

    n_rdma = 2 * (N_DEV - 1)
    return pl.pallas_call(
        body,
        out_shape=jax.ShapeDtypeStruct((B, Sq, D), jnp.float32),
        in_specs=[
            pl.BlockSpec(memory_space=pltpu.VMEM),
            pl.BlockSpec(memory_space=pltpu.VMEM),
            pl.BlockSpec(memory_space=pltpu.VMEM),
            pl.BlockSpec(memory_space=pltpu.HBM),
            pl.BlockSpec(memory_space=pltpu.HBM),
        ],
        out_specs=pl.BlockSpec(memory_space=pltpu.VMEM),
        scratch_shapes=[
            pltpu.VMEM((N_DEV, Sq, D), jnp.bfloat16),
            pltpu.VMEM((n_rdma, Sq, D), jnp.bfloat16),
            pltpu.VMEM((2, Skv, Hloc, Dh), jnp.float32),
            pltpu.VMEM((2, Skv, Hloc, Dh), jnp.float32),
            pltpu.SemaphoreType.DMA((2, 2)),
            pltpu.SemaphoreType.DMA((n_rdma,)),
            pltpu.SemaphoreType.DMA((n_rdma,)),
        ],
        compiler_params=pltpu.CompilerParams(
            collective_id=0,
            vmem_limit_bytes=100 * 1024 * 1024,
        ),
    )(x, Wq, Wo, K_ext, V_ext)


# device time: 59391 ns/iter; 1.9758x vs baseline; 1.0340x over previous
import jax
import jax.numpy as jnp
from jax import lax
from jax.experimental import pallas as pl
from jax.experimental.pallas import tpu as pltpu

N_DEV = 4
SCALE = 0.08838834764831843


def kernel(x, Wq, Wo, K_ext, V_ext):
    B, Sq, D = x.shape
    _, Skv, Hloc, Dh = K_ext.shape

    def body(x_ref, wq_ref, wo_ref, k_hbm, v_hbm, out_ref,
             send_buf, recv_buf, k_stage, v_stage,
             kv_sems, send_sems, recv_sems):
        my = lax.axis_index("i")
        left = lax.rem(my + N_DEV - 1, N_DEV)
        right = lax.rem(my + 1, N_DEV)

        barrier_sem = pltpu.get_barrier_semaphore()
        for nbr in (left, right):
            pl.semaphore_signal(barrier_sem, inc=1, device_id=(nbr,),
                                device_id_type=pl.DeviceIdType.MESH)
        pl.semaphore_wait(barrier_sem, 2)

        def kv_copies(b, slot):
            ck = pltpu.make_async_copy(
                k_hbm.at[b], k_stage.at[slot], kv_sems.at[slot, 0])
            cv = pltpu.make_async_copy(
                v_hbm.at[b], v_stage.at[slot], kv_sems.at[slot, 1])
            return ck, cv

        def ring_rdma(r, src_ref, to=None):
            return pltpu.make_async_remote_copy(
                src_ref=src_ref,
                dst_ref=recv_buf.at[r],
                send_sem=send_sems.at[r],
                recv_sem=recv_sems.at[r],
                device_id=(right if to is None else to,),
                device_id_type=pl.DeviceIdType.MESH,
            )

        wq = wq_ref[...].astype(jnp.bfloat16)
        wo = wo_ref[...].astype(jnp.bfloat16)

        def batch_at(t):
            return lax.rem(my - t + 2 * N_DEV, N_DEV)

        b0 = batch_at(0)
        ck, cv = kv_copies(b0, 0)
        ck.start()
        cv.start()

        ones_blk = jnp.ones((Skv, Dh), jnp.bfloat16)

        def compute_partial(b, slot):
            xb = x_ref[b].astype(jnp.bfloat16)
            qb = jnp.dot(xb, wq, preferred_element_type=jnp.float32)
            qb = (qb * SCALE).astype(jnp.bfloat16)
            kb = k_stage[slot].reshape(Skv, Hloc * Dh).astype(jnp.bfloat16)
            vb = v_stage[slot].reshape(Skv, Hloc * Dh).astype(jnp.bfloat16)
            heads = []
            for h in range(Hloc):
                qh = qb[:, h * Dh:(h + 1) * Dh]
                kh = kb[:, h * Dh:(h + 1) * Dh]
                vh = vb[:, h * Dh:(h + 1) * Dh]
                s = lax.dot_general(qh, kh, (((1,), (1,)), ((), ())),
                                    preferred_element_type=jnp.float32)
                p = jnp.exp(s.astype(jnp.bfloat16))
                vhe = jnp.concatenate([vh, ones_blk], axis=1)
                r = jnp.dot(p, vhe, preferred_element_type=jnp.float32)
                o = r[:, :Dh] / r[:, Dh:Dh + 1]
                heads.append(o.astype(jnp.bfloat16))
            ab = jnp.concatenate(heads, axis=1)
            return jnp.dot(ab, wo, preferred_element_type=jnp.float32)

        if B > 1:
            nb, nv = kv_copies(batch_at(1), 1)
            nb.start()
            nv.start()
        ck, cv = kv_copies(b0, 0)
        ck.wait()
        cv.wait()
        p0 = compute_partial(b0, 0)
        out_ref[b0] = p0
        send_buf[0] = p0.astype(jnp.bfloat16)
        rs = [None] * (N_DEV - 1)
        rs[0] = ring_rdma(0, send_buf.at[0])
        rs[0].start()

        for t in range(1, N_DEV):
            slot = t % 2
            if t + 1 < N_DEV:
                nk, nv = kv_copies(batch_at(t + 1), 1 - slot)
                nk.start()
                nv.start()
            ck, cv = kv_copies(batch_at(t), slot)
            ck.wait()
            cv.wait()
            bt = batch_at(t)
            pt = compute_partial(bt, slot)
            out_ref[bt] = pt
            rs[t - 1].wait_recv()
            acc = recv_buf[t - 1].astype(jnp.float32) + pt
            if t < N_DEV - 1:
                send_buf[t] = acc.astype(jnp.bfloat16)
                rs[t] = ring_rdma(t, send_buf.at[t])
                rs[t].start()
            else:
                out_ref[bt] = acc
                send_buf[N_DEV - 1] = acc.astype(jnp.bfloat16)

        ag0r = ring_rdma(3, send_buf.at[N_DEV - 1])
        ag0l = ring_rdma(4, send_buf.at[N_DEV - 1], to=left)
        ag0r.start()
        ag0l.start()
        ag0r.wait_recv()
        ag1 = ring_rdma(5, recv_buf.at[3])
        ag1.start()
        out_ref[my] = recv_buf[3].astype(jnp.float32)
        ag0l.wait_recv()
        out_ref[lax.rem(my + 2, N_DEV)] = recv_buf[4].astype(jnp.float32)
        ag1.wait_recv()
        out_ref[lax.rem(my + N_DEV - 1, N_DEV)] = recv_buf[5].astype(jnp.float32)
        ag = [ag0r, ag0l, ag1]

        for r, d in enumerate(rs + ag):
            d.wait_send()

    n_rdma = 2 * (N_DEV - 1)
    return pl.pallas_call(
        body,
        out_shape=jax.ShapeDtypeStruct((B, Sq, D), jnp.float32),
        in_specs=[
            pl.BlockSpec(memory_space=pltpu.VMEM),
            pl.BlockSpec(memory_space=pltpu.VMEM),
            pl.BlockSpec(memory_space=pltpu.VMEM),
            pl.BlockSpec(memory_space=pltpu.HBM),
            pl.BlockSpec(memory_space=pltpu.HBM),
        ],
        out_specs=pl.BlockSpec(memory_space=pltpu.VMEM),
        scratch_shapes=[
            pltpu.VMEM((N_DEV, Sq, D), jnp.bfloat16),
            pltpu.VMEM((n_rdma, Sq, D), jnp.bfloat16),
            pltpu.VMEM((2, Skv, Hloc, Dh), jnp.float32),
            pltpu.VMEM((2, Skv, Hloc, Dh), jnp.float32),
            pltpu.SemaphoreType.DMA((2, 2)),
            pltpu.SemaphoreType.DMA((n_rdma,)),
            pltpu.SemaphoreType.DMA((n_rdma,)),
        ],
        compiler_params=pltpu.CompilerParams(
            collective_id=0,
            vmem_limit_bytes=100 * 1024 * 1024,
        ),
    )(x, Wq, Wo, K_ext, V_ext)


# device time: 53844 ns/iter; 2.1794x vs baseline; 1.1030x over previous
import jax
import jax.numpy as jnp
from jax import lax
from jax.experimental import pallas as pl
from jax.experimental.pallas import tpu as pltpu

N_DEV = 4
SCALE = 0.08838834764831843


def kernel(x, Wq, Wo, K_ext, V_ext):
    B, Sq, D = x.shape
    _, Skv, Hloc, Dh = K_ext.shape

    def body(x_ref, wq_ref, wo_ref, k_hbm, v_hbm, out_ref,
             send_buf, recv_buf, k_stage, v_stage,
             kv_sems, send_sems, recv_sems):
        my = lax.axis_index("i")
        left = lax.rem(my + N_DEV - 1, N_DEV)
        right = lax.rem(my + 1, N_DEV)
        diag = lax.rem(my + 2, N_DEV)

        barrier_sem = pltpu.get_barrier_semaphore()
        for nbr in (left, right, diag):
            pl.semaphore_signal(barrier_sem, inc=1, device_id=(nbr,),
                                device_id_type=pl.DeviceIdType.MESH)
        pl.semaphore_wait(barrier_sem, 3)

        def kv_copies(b, slot):
            ck = pltpu.make_async_copy(
                k_hbm.at[b], k_stage.at[slot], kv_sems.at[slot, 0])
            cv = pltpu.make_async_copy(
                v_hbm.at[b], v_stage.at[slot], kv_sems.at[slot, 1])
            return ck, cv

        def ring_rdma(r, src_ref, to=None):
            return pltpu.make_async_remote_copy(
                src_ref=src_ref,
                dst_ref=recv_buf.at[r],
                send_sem=send_sems.at[r],
                recv_sem=recv_sems.at[r],
                device_id=(right if to is None else to,),
                device_id_type=pl.DeviceIdType.MESH,
            )

        wq = wq_ref[...].astype(jnp.bfloat16)
        wo = wo_ref[...].astype(jnp.bfloat16)

        _OFFS = (2, 1, 3, 0)

        def batch_at(t):
            return lax.rem(my + _OFFS[t], N_DEV)

        b0 = batch_at(0)
        ck, cv = kv_copies(b0, 0)
        ck.start()
        cv.start()

        ones_blk = jnp.ones((Skv, Dh), jnp.bfloat16)

        def compute_partial(b, slot):
            xb = x_ref[b].astype(jnp.bfloat16)
            qb = jnp.dot(xb, wq, preferred_element_type=jnp.float32)
            qb = (qb * SCALE).astype(jnp.bfloat16)
            kb = k_stage[slot].reshape(Skv, Hloc * Dh).astype(jnp.bfloat16)
            vb = v_stage[slot].reshape(Skv, Hloc * Dh).astype(jnp.bfloat16)
            heads = []
            for h in range(Hloc):
                qh = qb[:, h * Dh:(h + 1) * Dh]
                kh = kb[:, h * Dh:(h + 1) * Dh]
                vh = vb[:, h * Dh:(h + 1) * Dh]
                s = lax.dot_general(qh, kh, (((1,), (1,)), ((), ())),
                                    preferred_element_type=jnp.float32)
                p = jnp.exp(s.astype(jnp.bfloat16))
                vhe = jnp.concatenate([vh, ones_blk], axis=1)
                r = jnp.dot(p, vhe, preferred_element_type=jnp.float32)
                o = r[:, :Dh] / r[:, Dh:Dh + 1]
                heads.append(o.astype(jnp.bfloat16))
            ab = jnp.concatenate(heads, axis=1)
            return jnp.dot(ab, wo, preferred_element_type=jnp.float32)

        _TARGET = (diag, right, left)
        _DSLOT = (2, 0, 1)

        scatter = []
        for t in range(3):
            slot = t % 2
            if t + 1 < N_DEV:
                nk, nv = kv_copies(batch_at(t + 1), 1 - slot)
                nk.start()
                nv.start()
            ck, cv = kv_copies(batch_at(t), slot)
            ck.wait()
            cv.wait()
            pt = compute_partial(batch_at(t), slot)
            send_buf[t] = pt.astype(jnp.bfloat16)
            sd = ring_rdma(_DSLOT[t], send_buf.at[t], to=_TARGET[t])
            sd.start()
            scatter.append(sd)

        ck, cv = kv_copies(my, 1)
        ck.wait()
        cv.wait()
        pown = compute_partial(my, 1)
        for j in range(3):
            recv_only = ring_rdma(j, send_buf.at[0])
            recv_only.wait_recv()
        acc = (pown
               + recv_buf[0].astype(jnp.float32)
               + recv_buf[1].astype(jnp.float32)
               + recv_buf[2].astype(jnp.float32))
        out_ref[my] = acc
        send_buf[3] = acc.astype(jnp.bfloat16)

        ag0r = ring_rdma(3, send_buf.at[3])
        ag0l = ring_rdma(4, send_buf.at[3], to=left)
        ag0r.start()
        ag0l.start()
        ag0r.wait_recv()
        ag1 = ring_rdma(5, recv_buf.at[3])
        ag1.start()
        out_ref[left] = recv_buf[3].astype(jnp.float32)
        ag0l.wait_recv()
        out_ref[right] = recv_buf[4].astype(jnp.float32)
        ag1.wait_recv()
        out_ref[diag] = recv_buf[5].astype(jnp.float32)
        ag = [ag0r, ag0l, ag1]
        rs = scatter

        for r, d in enumerate(rs + ag):
            d.wait_send()

    n_rdma = 2 * (N_DEV - 1)
    return pl.pallas_call(
        body,
        out_shape=jax.ShapeDtypeStruct((B, Sq, D), jnp.float32),
        in_specs=[
            pl.BlockSpec(memory_space=pltpu.VMEM),
            pl.BlockSpec(memory_space=pltpu.VMEM),
            pl.BlockSpec(memory_space=pltpu.VMEM),
            pl.BlockSpec(memory_space=pltpu.HBM),
            pl.BlockSpec(memory_space=pltpu.HBM),
        ],
        out_specs=pl.BlockSpec(memory_space=pltpu.VMEM),
        scratch_shapes=[
            pltpu.VMEM((N_DEV, Sq, D), jnp.bfloat16),
            pltpu.VMEM((n_rdma, Sq, D), jnp.bfloat16),
            pltpu.VMEM((2, Skv, Hloc, Dh), jnp.float32),
            pltpu.VMEM((2, Skv, Hloc, Dh), jnp.float32),
            pltpu.SemaphoreType.DMA((2, 2)),
            pltpu.SemaphoreType.DMA((n_rdma,)),
            pltpu.SemaphoreType.DMA((n_rdma,)),
        ],
        compiler_params=pltpu.CompilerParams(
            collective_id=0,
            vmem_limit_bytes=100 * 1024 * 1024,
        ),
    )(x, Wq, Wo, K_ext, V_ext)


# device time: 52699 ns/iter; 2.2267x vs baseline; 1.0217x over previous
import jax
import jax.numpy as jnp
from jax import lax
from jax.experimental import pallas as pl
from jax.experimental.pallas import tpu as pltpu

N_DEV = 4
SCALE = 0.08838834764831843


def kernel(x, Wq, Wo, K_ext, V_ext):
    B, Sq, D = x.shape
    _, Skv, Hloc, Dh = K_ext.shape

    def body(x_ref, wq_ref, wo_ref, k_hbm, v_hbm, out_ref,
             send_buf, recv_buf, k_stage, v_stage,
             kv_sems, send_sems, recv_sems):
        my = lax.axis_index("i")
        left = lax.rem(my + N_DEV - 1, N_DEV)
        right = lax.rem(my + 1, N_DEV)
        diag = lax.rem(my + 2, N_DEV)

        barrier_sem = pltpu.get_barrier_semaphore()
        for nbr in (left, right, diag):
            pl.semaphore_signal(barrier_sem, inc=1, device_id=(nbr,),
                                device_id_type=pl.DeviceIdType.MESH)
        pl.semaphore_wait(barrier_sem, 3)

        def kv_copies(b, slot):
            ck = pltpu.make_async_copy(
                k_hbm.at[b], k_stage.at[slot], kv_sems.at[slot, 0])
            cv = pltpu.make_async_copy(
                v_hbm.at[b], v_stage.at[slot], kv_sems.at[slot, 1])
            return ck, cv

        def ring_rdma(r, src_ref, to=None):
            return pltpu.make_async_remote_copy(
                src_ref=src_ref,
                dst_ref=recv_buf.at[r],
                send_sem=send_sems.at[r],
                recv_sem=recv_sems.at[r],
                device_id=(right if to is None else to,),
                device_id_type=pl.DeviceIdType.MESH,
            )

        wq = wq_ref[...].astype(jnp.bfloat16)
        wo = wo_ref[...].astype(jnp.bfloat16)

        _OFFS = (2, 1, 3, 0)

        def batch_at(t):
            return lax.rem(my + _OFFS[t], N_DEV)

        b0 = batch_at(0)
        ck, cv = kv_copies(b0, 0)
        ck.start()
        cv.start()

        ones_blk = jnp.ones((Skv, Dh), jnp.bfloat16)

        def compute_partial(b, slot):
            xb = x_ref[b].astype(jnp.bfloat16)
            qb = jnp.dot(xb, wq, preferred_element_type=jnp.float32)
            qb = (qb * SCALE).astype(jnp.bfloat16)
            kb = k_stage[slot].reshape(Skv, Hloc * Dh).astype(jnp.bfloat16)
            vb = v_stage[slot].reshape(Skv, Hloc * Dh).astype(jnp.bfloat16)
            heads = []
            for h in range(Hloc):
                qh = qb[:, h * Dh:(h + 1) * Dh]
                kh = kb[:, h * Dh:(h + 1) * Dh]
                vh = vb[:, h * Dh:(h + 1) * Dh]
                s = lax.dot_general(qh, kh, (((1,), (1,)), ((), ())),
                                    preferred_element_type=jnp.float32)
                p = jnp.exp(s.astype(jnp.bfloat16))
                vhe = jnp.concatenate([vh, ones_blk], axis=1)
                r = jnp.dot(p, vhe, preferred_element_type=jnp.float32)
                o = r[:, :Dh] / r[:, Dh:Dh + 1]
                heads.append(o.astype(jnp.bfloat16))
            ab = jnp.concatenate(heads, axis=1)
            return jnp.dot(ab, wo, preferred_element_type=jnp.float32)

        _TARGET = (diag, right, left)
        _DSLOT = (2, 0, 1)

        scatter = []
        for t in range(3):
            slot = t % 2
            if t + 1 < N_DEV:
                nk, nv = kv_copies(batch_at(t + 1), 1 - slot)
                nk.start()
                nv.start()
            ck, cv = kv_copies(batch_at(t), slot)
            ck.wait()
            cv.wait()
            pt = compute_partial(batch_at(t), slot)
            send_buf[t] = pt.astype(jnp.bfloat16)
            sd = ring_rdma(_DSLOT[t], send_buf.at[t], to=_TARGET[t])
            sd.start()
            scatter.append(sd)

        ck, cv = kv_copies(my, 1)
        ck.wait()
        cv.wait()
        pown = compute_partial(my, 1)
        for j in range(3):
            recv_only = ring_rdma(j, send_buf.at[0])
            recv_only.wait_recv()
        acc = (pown
               + recv_buf[0].astype(jnp.float32)
               + recv_buf[1].astype(jnp.float32)
               + recv_buf[2].astype(jnp.float32))
        out_ref[my] = acc
        send_buf[3] = acc.astype(jnp.bfloat16)

        agr = ring_rdma(3, send_buf.at[3], to=right)
        agl = ring_rdma(4, send_buf.at[3], to=left)
        agd = ring_rdma(5, send_buf.at[3], to=diag)
        agr.start()
        agl.start()
        agd.start()
        agr.wait_recv()
        out_ref[left] = recv_buf[3].astype(jnp.float32)
        agl.wait_recv()
        out_ref[right] = recv_buf[4].astype(jnp.float32)
        agd.wait_recv()
        out_ref[diag] = recv_buf[5].astype(jnp.float32)
        ag = [agr, agl, agd]
        rs = scatter

        for r, d in enumerate(rs + ag):
            d.wait_send()

    n_rdma = 2 * (N_DEV - 1)
    return pl.pallas_call(
        body,
        out_shape=jax.ShapeDtypeStruct((B, Sq, D), jnp.float32),
        in_specs=[
            pl.BlockSpec(memory_space=pltpu.VMEM),
            pl.BlockSpec(memory_space=pltpu.VMEM),
            pl.BlockSpec(memory_space=pltpu.VMEM),
            pl.BlockSpec(memory_space=pltpu.HBM),
            pl.BlockSpec(memory_space=pltpu.HBM),
        ],
        out_specs=pl.BlockSpec(memory_space=pltpu.VMEM),
        scratch_shapes=[
            pltpu.VMEM((N_DEV, Sq, D), jnp.bfloat16),
            pltpu.VMEM((n_rdma, Sq, D), jnp.bfloat16),
            pltpu.VMEM((2, Skv, Hloc, Dh), jnp.float32),
            pltpu.VMEM((2, Skv, Hloc, Dh), jnp.float32),
            pltpu.SemaphoreType.DMA((2, 2)),
            pltpu.SemaphoreType.DMA((n_rdma,)),
            pltpu.SemaphoreType.DMA((n_rdma,)),
        ],
        compiler_params=pltpu.CompilerParams(
            collective_id=0,
            vmem_limit_bytes=100 * 1024 * 1024,
        ),
    )(x, Wq, Wo, K_ext, V_ext)


# device time: 51280 ns/iter; 2.2883x vs baseline; 1.0277x over previous
import jax
import jax.numpy as jnp
from jax import lax
from jax.experimental import pallas as pl
from jax.experimental.pallas import tpu as pltpu

N_DEV = 4
SCALE = 0.08838834764831843


def kernel(x, Wq, Wo, K_ext, V_ext):
    B, Sq, D = x.shape
    _, Skv, Hloc, Dh = K_ext.shape

    def body(x_ref, wq_ref, wo_ref, k_hbm, v_hbm, out_ref,
             send_buf, recv_buf, k_stage, v_stage,
             kv_sems, send_sems, recv_sems):
        my = lax.axis_index("i")
        left = lax.rem(my + N_DEV - 1, N_DEV)
        right = lax.rem(my + 1, N_DEV)
        diag = lax.rem(my + 2, N_DEV)

        barrier_sem = pltpu.get_barrier_semaphore()
        for nbr in (left, right, diag):
            pl.semaphore_signal(barrier_sem, inc=1, device_id=(nbr,),
                                device_id_type=pl.DeviceIdType.MESH)
        pl.semaphore_wait(barrier_sem, 3)

        def kv_copies(b, slot):
            ck = pltpu.make_async_copy(
                k_hbm.at[b], k_stage.at[slot], kv_sems.at[slot, 0])
            cv = pltpu.make_async_copy(
                v_hbm.at[b], v_stage.at[slot], kv_sems.at[slot, 1])
            return ck, cv

        def ring_rdma(r, src_ref, to=None):
            return pltpu.make_async_remote_copy(
                src_ref=src_ref,
                dst_ref=recv_buf.at[r],
                send_sem=send_sems.at[r],
                recv_sem=recv_sems.at[r],
                device_id=(right if to is None else to,),
                device_id_type=pl.DeviceIdType.MESH,
            )

        wq = wq_ref[...].astype(jnp.bfloat16)
        wo = wo_ref[...].astype(jnp.bfloat16)

        _OFFS = (2, 1, 3, 0)

        def batch_at(t):
            return lax.rem(my + _OFFS[t], N_DEV)

        b0 = batch_at(0)
        ck, cv = kv_copies(b0, 0)
        ck.start()
        cv.start()

        ones_blk = jnp.ones((Skv, Dh), jnp.bfloat16)

        def compute_partial(b, slot):
            kb = k_stage[slot].reshape(Skv, Hloc * Dh).astype(jnp.bfloat16)
            vb = v_stage[slot].reshape(Skv, Hloc * Dh).astype(jnp.bfloat16)
            return attn_rows(x_ref[b].astype(jnp.bfloat16), kb, vb)

        def attn_rows(xr, kb, vb):
            qb = jnp.dot(xr, wq, preferred_element_type=jnp.float32)
            qb = (qb * SCALE).astype(jnp.bfloat16)
            heads = []
            for h in range(Hloc):
                qh = qb[:, h * Dh:(h + 1) * Dh]
                kh = kb[:, h * Dh:(h + 1) * Dh]
                vh = vb[:, h * Dh:(h + 1) * Dh]
                s = lax.dot_general(qh, kh, (((1,), (1,)), ((), ())),
                                    preferred_element_type=jnp.float32)
                p = jnp.exp(s.astype(jnp.bfloat16))
                vhe = jnp.concatenate([vh, ones_blk], axis=1)
                r = jnp.dot(p, vhe, preferred_element_type=jnp.float32)
                o = r[:, :Dh] / r[:, Dh:Dh + 1]
                heads.append(o.astype(jnp.bfloat16))
            ab = jnp.concatenate(heads, axis=1)
            return jnp.dot(ab, wo, preferred_element_type=jnp.float32)

        _TARGET = (diag, right, left)
        _DSLOT = (2, 0, 1)
        Sq2 = Sq // 2

        def half_rdma(slot, h, sem, src_ref, to):
            return pltpu.make_async_remote_copy(
                src_ref=src_ref,
                dst_ref=recv_buf.at[slot, pl.ds(h * Sq2, Sq2)],
                send_sem=send_sems.at[sem],
                recv_sem=recv_sems.at[sem],
                device_id=(to,),
                device_id_type=pl.DeviceIdType.MESH,
            )

        dummy_src = send_buf.at[0, pl.ds(0, Sq2)]

        sends = []
        for t in range(3):
            slot = t % 2
            nk, nv = kv_copies(batch_at(t + 1), 1 - slot)
            nk.start()
            nv.start()
            ck, cv = kv_copies(batch_at(t), slot)
            ck.wait()
            cv.wait()
            pt = compute_partial(batch_at(t), slot)
            send_buf[t] = pt.astype(jnp.bfloat16)
            for h in range(2):
                sd = half_rdma(_DSLOT[t], h, 2 * t + h,
                               send_buf.at[t, pl.ds(h * Sq2, Sq2)],
                               _TARGET[t])
                sd.start()
                sends.append(sd)

        ck, cv = kv_copies(my, 1)
        ck.wait()
        cv.wait()
        kb = k_stage[1].reshape(Skv, Hloc * Dh).astype(jnp.bfloat16)
        vb = v_stage[1].reshape(Skv, Hloc * Dh).astype(jnp.bfloat16)
        xall = x_ref[my].astype(jnp.bfloat16)
        ag_targets = ((right, 3), (left, 4), (diag, 5))
        for h in range(2):
            rows = slice(h * Sq2, (h + 1) * Sq2)
            ph = attn_rows(xall[rows], kb, vb)
            for t in range(3):
                half_rdma(_DSLOT[t], h, 2 * t + h, dummy_src,
                          right).wait_recv()
            acc = (ph
                   + recv_buf[0, rows].astype(jnp.float32)
                   + recv_buf[1, rows].astype(jnp.float32)
                   + recv_buf[2, rows].astype(jnp.float32))
            out_ref[my, rows] = acc
            send_buf[3, rows] = acc.astype(jnp.bfloat16)
            for p, (tgt, dslot) in enumerate(ag_targets):
                d = half_rdma(dslot, h, 6 + 2 * p + h,
                              send_buf.at[3, pl.ds(h * Sq2, Sq2)], tgt)
                d.start()
                sends.append(d)

        for p, dslot in enumerate((3, 4, 5)):
            for h in range(2):
                half_rdma(dslot, h, 6 + 2 * p + h, dummy_src,
                          right).wait_recv()
        out_ref[left] = recv_buf[3].astype(jnp.float32)
        out_ref[right] = recv_buf[4].astype(jnp.float32)
        out_ref[diag] = recv_buf[5].astype(jnp.float32)

        for d in sends:
            d.wait_send()

    n_rdma = 2 * (N_DEV - 1)
    return pl.pallas_call(
        body,
        out_shape=jax.ShapeDtypeStruct((B, Sq, D), jnp.float32),
        in_specs=[
            pl.BlockSpec(memory_space=pltpu.VMEM),
            pl.BlockSpec(memory_space=pltpu.VMEM),
            pl.BlockSpec(memory_space=pltpu.VMEM),
            pl.BlockSpec(memory_space=pltpu.HBM),
            pl.BlockSpec(memory_space=pltpu.HBM),
        ],
        out_specs=pl.BlockSpec(memory_space=pltpu.VMEM),
        scratch_shapes=[
            pltpu.VMEM((N_DEV, Sq, D), jnp.bfloat16),
            pltpu.VMEM((n_rdma, Sq, D), jnp.bfloat16),
            pltpu.VMEM((2, Skv, Hloc, Dh), jnp.float32),
            pltpu.VMEM((2, Skv, Hloc, Dh), jnp.float32),
            pltpu.SemaphoreType.DMA((2, 2)),
            pltpu.SemaphoreType.DMA((2 * n_rdma,)),
            pltpu.SemaphoreType.DMA((2 * n_rdma,)),
        ],
        compiler_params=pltpu.CompilerParams(
            collective_id=0,
            vmem_limit_bytes=100 * 1024 * 1024,
        ),
    )(x, Wq, Wo, K_ext, V_ext)


# device time: 51130 ns/iter; 2.2950x vs baseline; 1.0029x over previous
import jax
import jax.numpy as jnp
from jax import lax
from jax.experimental import pallas as pl
from jax.experimental.pallas import tpu as pltpu

N_DEV = 4
SCALE = 0.08838834764831843


def kernel(x, Wq, Wo, K_ext, V_ext):
    B, Sq, D = x.shape
    _, Skv, Hloc, Dh = K_ext.shape

    def body(x_ref, wq_ref, wo_ref, k_hbm, v_hbm, out_ref,
             send_buf, recv_buf, k_stage, v_stage,
             kv_sems, send_sems, recv_sems):
        my = lax.axis_index("i")
        left = lax.rem(my + N_DEV - 1, N_DEV)
        right = lax.rem(my + 1, N_DEV)
        diag = lax.rem(my + 2, N_DEV)

        barrier_sem = pltpu.get_barrier_semaphore()
        for nbr in (left, right, diag):
            pl.semaphore_signal(barrier_sem, inc=1, device_id=(nbr,),
                                device_id_type=pl.DeviceIdType.MESH)
        pl.semaphore_wait(barrier_sem, 3)

        def kv_copies(b, slot):
            ck = pltpu.make_async_copy(
                k_hbm.at[b], k_stage.at[slot], kv_sems.at[slot, 0])
            cv = pltpu.make_async_copy(
                v_hbm.at[b], v_stage.at[slot], kv_sems.at[slot, 1])
            return ck, cv

        def ring_rdma(r, src_ref, to=None):
            return pltpu.make_async_remote_copy(
                src_ref=src_ref,
                dst_ref=recv_buf.at[r],
                send_sem=send_sems.at[r],
                recv_sem=recv_sems.at[r],
                device_id=(right if to is None else to,),
                device_id_type=pl.DeviceIdType.MESH,
            )

        wq = wq_ref[...].astype(jnp.bfloat16)
        wo = wo_ref[...].astype(jnp.bfloat16)

        _OFFS = (2, 1, 3, 0)

        def batch_at(t):
            return lax.rem(my + _OFFS[t], N_DEV)

        b0 = batch_at(0)
        ck, cv = kv_copies(b0, 0)
        ck.start()
        cv.start()

        ones_blk = jnp.ones((Skv, Dh), jnp.bfloat16)

        def compute_partial(b, slot):
            kb = k_stage[slot].reshape(Skv, Hloc * Dh).astype(jnp.bfloat16)
            vb = v_stage[slot].reshape(Skv, Hloc * Dh).astype(jnp.bfloat16)
            return attn_rows(x_ref[b].astype(jnp.bfloat16), kb, vb)

        def attn_rows(xr, kb, vb):
            qb = jnp.dot(xr, wq, preferred_element_type=jnp.float32)
            qb = (qb * SCALE).astype(jnp.bfloat16)
            heads = []
            for h in range(Hloc):
                qh = qb[:, h * Dh:(h + 1) * Dh]
                kh = kb[:, h * Dh:(h + 1) * Dh]
                vh = vb[:, h * Dh:(h + 1) * Dh]
                s = lax.dot_general(qh, kh, (((1,), (1,)), ((), ())),
                                    preferred_element_type=jnp.float32)
                p = jnp.exp(s.astype(jnp.bfloat16))
                vhe = jnp.concatenate([vh, ones_blk], axis=1)
                r = jnp.dot(p, vhe, preferred_element_type=jnp.float32)
                o = r[:, :Dh] / r[:, Dh:Dh + 1]
                heads.append(o.astype(jnp.bfloat16))
            ab = jnp.concatenate(heads, axis=1)
            return jnp.dot(ab, wo, preferred_element_type=jnp.float32)

        _TARGET = (diag, right, left)
        _DSLOT = (2, 0, 1)
        Sq2 = Sq // 2

        def half_rdma(slot, h, sem, src_ref, to):
            return pltpu.make_async_remote_copy(
                src_ref=src_ref,
                dst_ref=recv_buf.at[slot, pl.ds(h * Sq2, Sq2)],
                send_sem=send_sems.at[sem],
                recv_sem=recv_sems.at[sem],
                device_id=(to,),
                device_id_type=pl.DeviceIdType.MESH,
            )

        dummy_src = send_buf.at[0, pl.ds(0, Sq2)]

        sends = []
        for t in range(3):
            slot = t % 2
            nk, nv = kv_copies(batch_at(t + 1), 1 - slot)
            nk.start()
            nv.start()
            ck, cv = kv_copies(batch_at(t), slot)
            ck.wait()
            cv.wait()
            pt = compute_partial(batch_at(t), slot)
            send_buf[t] = pt.astype(jnp.bfloat16)
            for h in range(2):
                sd = half_rdma(_DSLOT[t], h, 2 * t + h,
                               send_buf.at[t, pl.ds(h * Sq2, Sq2)],
                               _TARGET[t])
                sd.start()
                sends.append(sd)

        ck, cv = kv_copies(my, 1)
        ck.wait()
        cv.wait()
        kb = k_stage[1].reshape(Skv, Hloc * Dh).astype(jnp.bfloat16)
        vb = v_stage[1].reshape(Skv, Hloc * Dh).astype(jnp.bfloat16)
        xall = x_ref[my].astype(jnp.bfloat16)
        ag_targets = ((right, 3), (left, 4), (diag, 5))
        for h in range(2):
            rows = slice(h * Sq2, (h + 1) * Sq2)
            ph = attn_rows(xall[rows], kb, vb)
            for t in range(3):
                half_rdma(_DSLOT[t], h, 2 * t + h, dummy_src,
                          right).wait_recv()
            acc = (ph
                   + recv_buf[0, rows].astype(jnp.float32)
                   + recv_buf[1, rows].astype(jnp.float32)
                   + recv_buf[2, rows].astype(jnp.float32))
            out_ref[my, rows] = acc
            send_buf[3, rows] = acc.astype(jnp.bfloat16)
            for p, (tgt, dslot) in enumerate(ag_targets):
                d = half_rdma(dslot, h, 6 + 2 * p + h,
                              send_buf.at[3, pl.ds(h * Sq2, Sq2)], tgt)
                d.start()
                sends.append(d)

        chunk_of = {3: left, 4: right, 5: diag}
        for h in range(2):
            rows = slice(h * Sq2, (h + 1) * Sq2)
            for p, dslot in enumerate((3, 4, 5)):
                half_rdma(dslot, h, 6 + 2 * p + h, dummy_src,
                          right).wait_recv()
                out_ref[chunk_of[dslot], rows] = (
                    recv_buf[dslot, rows].astype(jnp.float32))

        for d in sends:
            d.wait_send()

    n_rdma = 2 * (N_DEV - 1)
    return pl.pallas_call(
        body,
        out_shape=jax.ShapeDtypeStruct((B, Sq, D), jnp.float32),
        in_specs=[
            pl.BlockSpec(memory_space=pltpu.VMEM),
            pl.BlockSpec(memory_space=pltpu.VMEM),
            pl.BlockSpec(memory_space=pltpu.VMEM),
            pl.BlockSpec(memory_space=pltpu.HBM),
            pl.BlockSpec(memory_space=pltpu.HBM),
        ],
        out_specs=pl.BlockSpec(memory_space=pltpu.VMEM),
        scratch_shapes=[
            pltpu.VMEM((N_DEV, Sq, D), jnp.bfloat16),
            pltpu.VMEM((n_rdma, Sq, D), jnp.bfloat16),
            pltpu.VMEM((2, Skv, Hloc, Dh), jnp.float32),
            pltpu.VMEM((2, Skv, Hloc, Dh), jnp.float32),
            pltpu.SemaphoreType.DMA((2, 2)),
            pltpu.SemaphoreType.DMA((2 * n_rdma,)),
            pltpu.SemaphoreType.DMA((2 * n_rdma,)),
        ],
        compiler_params=pltpu.CompilerParams(
            collective_id=0,
            vmem_limit_bytes=100 * 1024 * 1024,
        ),
    )(x, Wq, Wo, K_ext, V_ext)
